# baseline (speedup 1.0000x reference)
import functools

import jax
import jax.numpy as jnp
from jax import lax
from jax.experimental import pallas as pl
from jax.experimental.pallas import tpu as pltpu
from jax.experimental.pallas import tpu_sc as plsc

_OH, _OW = 7, 7
_SCALE = 0.0625
_NLVL = 4
_NTBL = _NLVL * _NLVL

_NP = 1024
_NSC = 32
_RPT = _NP // _NSC
_NCELL = _OH * _OW
_GRP = 13
_IDXW = 112
_NROW = 2 * _IDXW
_OPAD = 56

_INTERPRET = False


def _rowtab_body(f_ref, r_ref, *, H, W, C):
    a = f_ref[0]
    r_ref[0, 0] = a
    for l in range(1, _NLVL):
        d = 1 << (l - 1)
        prev = r_ref[l - 1, 0]
        shifted = jnp.concatenate(
            [prev[d:], jnp.broadcast_to(prev[H - 1:], (d, W, C))], axis=0)
        r_ref[l, 0] = jnp.maximum(prev, shifted)


def _coltab_body(r_ref, t_ref, *, H, W, C):
    q = r_ref[0, 0]
    t_ref[0, 0, 0] = q
    for m in range(1, _NLVL):
        d = 1 << (m - 1)
        shifted = jnp.concatenate(
            [q[:, d:], jnp.broadcast_to(q[:, W - 1:], (H, d, C))], axis=1)
        q = jnp.maximum(q, shifted)
        t_ref[0, m, 0] = q


def _build_tables(feats):
    B, H, W, C = feats.shape
    rows = pl.pallas_call(
        functools.partial(_rowtab_body, H=H, W=W, C=C),
        grid=(B,),
        in_specs=[pl.BlockSpec((1, H, W, C), lambda b: (b, 0, 0, 0))],
        out_specs=pl.BlockSpec((_NLVL, 1, H, W, C),
                               lambda b: (0, b, 0, 0, 0)),
        out_shape=jax.ShapeDtypeStruct((_NLVL, B, H, W, C), jnp.float32),
        interpret=_INTERPRET,
    )(feats)
    return pl.pallas_call(
        functools.partial(_coltab_body, H=H, W=W, C=C),
        grid=(_NLVL, B),
        in_specs=[pl.BlockSpec((1, 1, H, W, C),
                               lambda l, b: (l, b, 0, 0, 0))],
        out_specs=pl.BlockSpec((1, _NLVL, 1, H, W, C),
                               lambda l, b: (l, 0, b, 0, 0, 0)),
        out_shape=jax.ShapeDtypeStruct((_NLVL, _NLVL, B, H, W, C),
                                       jnp.float32),
        interpret=_INTERPRET,
    )(rows)


def _make_sc_kernel(C, W, BHW):
    mesh = plsc.VectorSubcoreMesh(core_axis_name="c", subcore_axis_name="s")

    @functools.partial(
        pl.kernel,
        mesh=mesh,
        compiler_params=pltpu.CompilerParams(needs_layout_passes=False),
        out_type=jax.ShapeDtypeStruct((_NP * _OPAD * C,), jnp.float32),
        scratch_types=[
            pltpu.VMEM((_RPT * 16,), jnp.int32),
            pltpu.VMEM((2, _IDXW), jnp.int32),
            pltpu.VMEM((2, _IDXW), jnp.int32),
            pltpu.VMEM((_NROW, C), jnp.float32),
            pltpu.VMEM((_NROW, C), jnp.float32),
            pltpu.VMEM((_OPAD * C,), jnp.float32),
            pltpu.SemaphoreType.DMA,
            pltpu.SemaphoreType.DMA,
        ],
    )
    def sc_kernel(table_hbm, params_hbm, out_hbm,
                  pv, idx0, idx1, dat0, dat1, outv, sem0, sem1):
        info_nc = 2
        wid = lax.axis_index("s") * info_nc + lax.axis_index("c")
        r0 = wid * _RPT
        pltpu.sync_copy(params_hbm.at[pl.ds(r0 * 16, _RPT * 16)], pv)

        lane = lax.iota(jnp.int32, 16)
        corner = lane & 3
        want_hi_row = (corner & 2) != 0
        want_hi_col = (corner & 1) != 0

        def gen(r, idxbuf):
            pbase = r * 16

            def bcast(i):
                return plsc.load_gather(pv, [lane * 0 + (pbase + i)])

            basev = bcast(0)
            y1v = bcast(1)
            hv = bcast(2)
            x1v = bcast(3)
            wv = bcast(4)
            for g in range(_GRP):
                cell = jnp.minimum(g * 4 + (lane >> 2), _NCELL - 1)
                j = cell // _OW
                k = cell - j * _OW
                rs = y1v + (j * hv) // _OH
                re = y1v + ((j + 1) * hv + (_OH - 1)) // _OH
                rlen = re - rs
                cs = x1v + (k * wv) // _OW
                ce = x1v + ((k + 1) * wv + (_OW - 1)) // _OW
                clen = ce - cs
                lv = ((rlen >= 2).astype(jnp.int32)
                      + (rlen >= 4).astype(jnp.int32)
                      + (rlen >= 8).astype(jnp.int32))
                mv = ((clen >= 2).astype(jnp.int32)
                      + (clen >= 4).astype(jnp.int32)
                      + (clen >= 8).astype(jnp.int32))
                ya = jnp.where(want_hi_row, re - (1 << lv), rs)
                xa = jnp.where(want_hi_col, ce - (1 << mv), cs)
                t = lv * _NLVL + mv
                flat = t * BHW + basev + ya * W + xa
                idxbuf[g // 7, pl.ds((g % 7) * 16, 16)] = flat
            if _GRP % 7 != 0:
                for g in range(_GRP, 14):
                    idxbuf[g // 7, pl.ds((g % 7) * 16, 16)] = (
                        jnp.zeros((16,), jnp.int32))

        def fire(idxbuf, datbuf, sem):
            c0 = pltpu.async_copy(
                table_hbm.at[idxbuf.at[0]], datbuf.at[pl.ds(0, _IDXW)], sem)
            c1 = pltpu.async_copy(
                table_hbm.at[idxbuf.at[1]],
                datbuf.at[pl.ds(_IDXW, _IDXW)], sem)
            return c0, c1

        def reduce_out(r, datbuf):
            def cell_body(c, _):
                g = c // 4
                rb = (g // 7) * _IDXW + (g % 7) * 16 + (c - g * 4) * 4
                for ch in range(C // 16):
                    o = ch * 16
                    v01 = jnp.maximum(datbuf[rb, pl.ds(o, 16)],
                                      datbuf[rb + 1, pl.ds(o, 16)])
                    v23 = jnp.maximum(datbuf[rb + 2, pl.ds(o, 16)],
                                      datbuf[rb + 3, pl.ds(o, 16)])
                    outv[pl.ds(c * C + o, 16)] = jnp.maximum(v01, v23)
                return 0
            lax.fori_loop(0, _NCELL, cell_body, 0)
            pltpu.sync_copy(
                outv, out_hbm.at[pl.ds((r0 + r) * (_OPAD * C), _OPAD * C)])

        def roi_body(r, _):
            gen(r, idx0)
            for cp in fire(idx0, dat0, sem0):
                cp.wait()
            reduce_out(r, dat0)
            return 0

        lax.fori_loop(0, _RPT, roi_body, 0)
        del idx1, dat1, sem1

    return sc_kernel


def kernel(features, rois):
    B, C, H, W = features.shape
    N = rois.shape[0]

    feats = jnp.transpose(features, (0, 2, 3, 1))
    table = _build_tables(feats)
    table2d = table.reshape(_NTBL * B * H * W, C)

    bidx = rois[:, 0].astype(jnp.int32)
    coords = (rois[:, 1:5] * _SCALE).astype(jnp.int32)
    x1 = jnp.clip(coords[:, 0], 0, W - 1)
    y1 = jnp.clip(coords[:, 1], 0, H - 1)
    x2 = jnp.clip(coords[:, 2], 0, W - 1)
    y2 = jnp.clip(coords[:, 3], 0, H - 1)
    h = y2 - y1 + 1
    w = x2 - x1 + 1
    base = bidx * (H * W)

    cols = jnp.stack(
        [base, y1, h, x1, w] + [jnp.zeros_like(base)] * 11, axis=1)
    pad = jnp.tile(jnp.array([[0, 0, 1, 0, 1] + [0] * 11], jnp.int32),
                   (_NP - N, 1))
    params = jnp.concatenate([cols, pad], axis=0).reshape(_NP * 16)

    sc = _make_sc_kernel(C, W, B * H * W)
    out_flat = sc(table2d, params)

    out = out_flat.reshape(_NP, _OPAD, C)[:N, :_NCELL]
    out = out.reshape(N, _OH, _OW, C)
    return jnp.transpose(out, (0, 3, 1, 2))


# speedup vs baseline: 1.0419x; 1.0419x over previous
import functools

import jax
import jax.numpy as jnp
from jax import lax
from jax.experimental import pallas as pl
from jax.experimental.pallas import tpu as pltpu
from jax.experimental.pallas import tpu_sc as plsc

_OH, _OW = 7, 7
_SCALE = 0.0625
_NLVL = 4
_NTBL = _NLVL * _NLVL

_NP = 1024
_NSC = 32
_RPT = _NP // _NSC
_NCELL = _OH * _OW
_GRP = 13
_IDXW = 112
_NROW = 2 * _IDXW
_OPAD = 56

_INTERPRET = False


def _rowtab_body(f_ref, r_ref, *, H, W, C):
    a = f_ref[0]
    r_ref[0, 0] = a
    for l in range(1, _NLVL):
        d = 1 << (l - 1)
        prev = r_ref[l - 1, 0]
        shifted = jnp.concatenate(
            [prev[d:], jnp.broadcast_to(prev[H - 1:], (d, W, C))], axis=0)
        r_ref[l, 0] = jnp.maximum(prev, shifted)


def _coltab_body(r_ref, t_ref, *, H, W, C):
    q = r_ref[0, 0]
    t_ref[0, 0, 0] = q
    for m in range(1, _NLVL):
        d = 1 << (m - 1)
        shifted = jnp.concatenate(
            [q[:, d:], jnp.broadcast_to(q[:, W - 1:], (H, d, C))], axis=1)
        q = jnp.maximum(q, shifted)
        t_ref[0, m, 0] = q


def _build_tables(feats):
    B, H, W, C = feats.shape
    rows = pl.pallas_call(
        functools.partial(_rowtab_body, H=H, W=W, C=C),
        grid=(B,),
        in_specs=[pl.BlockSpec((1, H, W, C), lambda b: (b, 0, 0, 0))],
        out_specs=pl.BlockSpec((_NLVL, 1, H, W, C),
                               lambda b: (0, b, 0, 0, 0)),
        out_shape=jax.ShapeDtypeStruct((_NLVL, B, H, W, C), jnp.float32),
        interpret=_INTERPRET,
    )(feats)
    return pl.pallas_call(
        functools.partial(_coltab_body, H=H, W=W, C=C),
        grid=(_NLVL, B),
        in_specs=[pl.BlockSpec((1, 1, H, W, C),
                               lambda l, b: (l, b, 0, 0, 0))],
        out_specs=pl.BlockSpec((1, _NLVL, 1, H, W, C),
                               lambda l, b: (l, 0, b, 0, 0, 0)),
        out_shape=jax.ShapeDtypeStruct((_NLVL, _NLVL, B, H, W, C),
                                       jnp.float32),
        interpret=_INTERPRET,
    )(rows)


def _make_sc_kernel(C, W, BHW):
    mesh = plsc.VectorSubcoreMesh(core_axis_name="c", subcore_axis_name="s")

    @functools.partial(
        pl.kernel,
        mesh=mesh,
        compiler_params=pltpu.CompilerParams(needs_layout_passes=False),
        out_type=jax.ShapeDtypeStruct((_NP * _OPAD * C,), jnp.float32),
        scratch_types=[
            pltpu.VMEM((_RPT * 16,), jnp.int32),
            pltpu.VMEM((2, _IDXW), jnp.int32),
            pltpu.VMEM((2, _IDXW), jnp.int32),
            pltpu.VMEM((_NROW, C), jnp.float32),
            pltpu.VMEM((_NROW, C), jnp.float32),
            pltpu.VMEM((_OPAD * C,), jnp.float32),
            pltpu.SemaphoreType.DMA,
            pltpu.SemaphoreType.DMA,
        ],
    )
    def sc_kernel(table_hbm, params_hbm, out_hbm,
                  pv, idx0, idx1, dat0, dat1, outv, sem0, sem1):
        info_nc = 2
        wid = lax.axis_index("s") * info_nc + lax.axis_index("c")
        r0 = wid * _RPT
        pltpu.sync_copy(params_hbm.at[pl.ds(r0 * 16, _RPT * 16)], pv)

        lane = lax.iota(jnp.int32, 16)
        corner = lane & 3
        want_hi_row = (corner & 2) != 0
        want_hi_col = (corner & 1) != 0

        def gen(r, idxbuf):
            pbase = r * 16

            def bcast(i):
                return plsc.load_gather(pv, [lane * 0 + (pbase + i)])

            basev = bcast(0)
            y1v = bcast(1)
            hv = bcast(2)
            x1v = bcast(3)
            wv = bcast(4)
            sub = lane >> 2

            def sel4(vals):
                return jnp.where(
                    sub == 0, vals[0],
                    jnp.where(sub == 1, vals[1],
                              jnp.where(sub == 2, vals[2], vals[3])))

            def fd7(a):
                return (a * 9363) >> 16

            for g in range(_GRP):
                cells = [min(g * 4 + s, _NCELL - 1) for s in range(4)]
                j = sel4([c // _OW for c in cells])
                k = sel4([c % _OW for c in cells])
                rs = y1v + fd7(j * hv)
                re = y1v + fd7((j + 1) * hv + (_OH - 1))
                rlen = re - rs
                cs = x1v + fd7(k * wv)
                ce = x1v + fd7((k + 1) * wv + (_OW - 1))
                clen = ce - cs
                lv = ((rlen >= 2).astype(jnp.int32)
                      + (rlen >= 4).astype(jnp.int32)
                      + (rlen >= 8).astype(jnp.int32))
                mv = ((clen >= 2).astype(jnp.int32)
                      + (clen >= 4).astype(jnp.int32)
                      + (clen >= 8).astype(jnp.int32))
                ya = jnp.where(want_hi_row, re - (1 << lv), rs)
                xa = jnp.where(want_hi_col, ce - (1 << mv), cs)
                t = lv * _NLVL + mv
                flat = t * BHW + basev + ya * W + xa
                idxbuf[g // 7, pl.ds((g % 7) * 16, 16)] = flat
            if _GRP % 7 != 0:
                for g in range(_GRP, 14):
                    idxbuf[g // 7, pl.ds((g % 7) * 16, 16)] = (
                        jnp.zeros((16,), jnp.int32))

        def fire(idxbuf, datbuf, sem):
            c0 = pltpu.async_copy(
                table_hbm.at[idxbuf.at[0]], datbuf.at[pl.ds(0, _IDXW)], sem)
            c1 = pltpu.async_copy(
                table_hbm.at[idxbuf.at[1]],
                datbuf.at[pl.ds(_IDXW, _IDXW)], sem)
            return c0, c1

        def reduce_out(r, datbuf):
            def cell_body(c, _):
                g = c >> 2
                q = (g >= 7).astype(jnp.int32)
                rb = q * _IDXW + ((g - 7 * q) << 4) + ((c - (g << 2)) << 2)
                for ch in range(C // 16):
                    o = ch * 16
                    v01 = jnp.maximum(datbuf[rb, pl.ds(o, 16)],
                                      datbuf[rb + 1, pl.ds(o, 16)])
                    v23 = jnp.maximum(datbuf[rb + 2, pl.ds(o, 16)],
                                      datbuf[rb + 3, pl.ds(o, 16)])
                    outv[pl.ds(c * C + o, 16)] = jnp.maximum(v01, v23)
                return 0
            lax.fori_loop(0, _NCELL, cell_body, 0)
            pltpu.sync_copy(
                outv, out_hbm.at[pl.ds((r0 + r) * (_OPAD * C), _OPAD * C)])

        def pair_body(i, _):
            ra = 2 * i
            rb = ra + 1
            gen(ra, idx0)
            a = fire(idx0, dat0, sem0)
            gen(rb, idx1)
            b = fire(idx1, dat1, sem1)
            for cp in a:
                cp.wait()
            reduce_out(ra, dat0)
            for cp in b:
                cp.wait()
            reduce_out(rb, dat1)
            return 0

        lax.fori_loop(0, _RPT // 2, pair_body, 0)

    return sc_kernel


def kernel(features, rois):
    B, C, H, W = features.shape
    N = rois.shape[0]

    feats = jnp.transpose(features, (0, 2, 3, 1))
    table = _build_tables(feats)
    table2d = table.reshape(_NTBL * B * H * W, C)

    bidx = rois[:, 0].astype(jnp.int32)
    coords = (rois[:, 1:5] * _SCALE).astype(jnp.int32)
    x1 = jnp.clip(coords[:, 0], 0, W - 1)
    y1 = jnp.clip(coords[:, 1], 0, H - 1)
    x2 = jnp.clip(coords[:, 2], 0, W - 1)
    y2 = jnp.clip(coords[:, 3], 0, H - 1)
    h = y2 - y1 + 1
    w = x2 - x1 + 1
    base = bidx * (H * W)

    cols = jnp.stack(
        [base, y1, h, x1, w] + [jnp.zeros_like(base)] * 11, axis=1)
    pad = jnp.tile(jnp.array([[0, 0, 1, 0, 1] + [0] * 11], jnp.int32),
                   (_NP - N, 1))
    params = jnp.concatenate([cols, pad], axis=0).reshape(_NP * 16)

    sc = _make_sc_kernel(C, W, B * H * W)
    out_flat = sc(table2d, params)

    out = out_flat.reshape(_NP, _OPAD, C)[:N, :_NCELL]
    out = out.reshape(N, _OH, _OW, C)
    return jnp.transpose(out, (0, 3, 1, 2))
